# baseline (speedup 1.0000x reference)
import functools

import jax
import jax.numpy as jnp
from jax import lax
from jax.experimental import pallas as pl
from jax.experimental.pallas import tpu as pltpu
from jax.experimental.pallas import tpu_sc as plsc

D_MODEL = 1024
D_FF = 2048
N_EXPERTS = 24
TOP_K = 3
EGYPTIAN = (1.0 / 2.0, 1.0 / 3.0, 1.0 / 6.0)

_T = 128
_NW = 32
_CH = 96


def _gather_rows(table, idx):
    B = idx.shape[0]
    Dm = table.shape[1]
    bpw = B // _NW
    assert bpw % _CH == 0
    mesh = plsc.VectorSubcoreMesh(core_axis_name="c", subcore_axis_name="s")

    @functools.partial(
        pl.kernel,
        out_type=jax.ShapeDtypeStruct((B, Dm), jnp.float32),
        mesh=mesh,
        scratch_types=[
            pltpu.VMEM((_CH,), jnp.int32),
            pltpu.VMEM((_CH, Dm), jnp.float32),
            pltpu.SemaphoreType.DMA,
        ],
    )
    def gather_kernel(table_hbm, idx_hbm, out_hbm, idx_v, rows_v, sem):
        wid = lax.axis_index("s") * 2 + lax.axis_index("c")
        base = wid * bpw
        for c in range(bpw // _CH):
            off = base + c * _CH
            pltpu.sync_copy(idx_hbm.at[pl.ds(off, _CH)], idx_v)
            pltpu.async_copy(table_hbm.at[idx_v], rows_v, sem).wait()
            pltpu.sync_copy(rows_v, out_hbm.at[pl.ds(off, _CH)])

    return gather_kernel(table, idx)


def _ffn_body(iexp_ref, rid_ref, rexp_ref, msc_ref,
              xg_ref, w1_hbm, b1_ref, w2_hbm, b2_ref, wrow_ref, y_ref,
              w1buf, w2buf, w1b, w2b, sem1, sem2):
    i = pl.program_id(0)
    n_used = msc_ref[0]
    n_runs = msc_ref[1]

    @pl.when(i < n_used)
    def _():
        r = rid_ref[i]
        slot = lax.rem(r, 2)
        is_first = (i == 0) | (rid_ref[jnp.maximum(i - 1, 0)] != r)

        @pl.when(is_first)
        def _():
            @pl.when(i == 0)
            def _():
                e0 = rexp_ref[0]
                pltpu.make_async_copy(
                    w1_hbm.at[e0], w1buf.at[0], sem1.at[0]).start()
                pltpu.make_async_copy(
                    w2_hbm.at[e0], w2buf.at[0], sem2.at[0]).start()

            e = rexp_ref[r]
            pltpu.make_async_copy(
                w1_hbm.at[e], w1buf.at[slot], sem1.at[slot]).wait()
            pltpu.make_async_copy(
                w2_hbm.at[e], w2buf.at[slot], sem2.at[slot]).wait()

            @pl.when(r + 1 < n_runs)
            def _():
                en = rexp_ref[r + 1]
                ns = 1 - slot
                pltpu.make_async_copy(
                    w1_hbm.at[en], w1buf.at[ns], sem1.at[ns]).start()
                pltpu.make_async_copy(
                    w2_hbm.at[en], w2buf.at[ns], sem2.at[ns]).start()

            w1b[...] = w1buf[slot].astype(jnp.bfloat16)
            w2b[...] = w2buf[slot].astype(jnp.bfloat16)

        h = jnp.dot(xg_ref[...].astype(jnp.bfloat16), w1b[...],
                    preferred_element_type=jnp.float32)
        h = jnp.maximum(h + b1_ref[0], 0.0)
        y = jnp.dot(h.astype(jnp.bfloat16), w2b[...],
                    preferred_element_type=jnp.float32)
        y_ref[...] = (y + b2_ref[0]) * wrow_ref[...]


def _ffn(xg, W1, b1, W2, b2, w_rows, item_expert, run_id, run_expert, msc):
    rows = xg.shape[0]
    grid_spec = pltpu.PrefetchScalarGridSpec(
        num_scalar_prefetch=4,
        grid=(rows // _T,),
        in_specs=[
            pl.BlockSpec((_T, D_MODEL), lambda i, ie, ri, re, ms: (i, 0)),
            pl.BlockSpec(memory_space=pltpu.MemorySpace.HBM),
            pl.BlockSpec((1, 1, D_FF), lambda i, ie, ri, re, ms: (ie[i], 0, 0)),
            pl.BlockSpec(memory_space=pltpu.MemorySpace.HBM),
            pl.BlockSpec((1, 1, D_MODEL),
                         lambda i, ie, ri, re, ms: (ie[i], 0, 0)),
            pl.BlockSpec((_T, 1), lambda i, ie, ri, re, ms: (i, 0)),
        ],
        out_specs=pl.BlockSpec((_T, D_MODEL), lambda i, ie, ri, re, ms: (i, 0)),
        scratch_shapes=[
            pltpu.VMEM((2, D_MODEL, D_FF), jnp.float32),
            pltpu.VMEM((2, D_FF, D_MODEL), jnp.float32),
            pltpu.VMEM((D_MODEL, D_FF), jnp.bfloat16),
            pltpu.VMEM((D_FF, D_MODEL), jnp.bfloat16),
            pltpu.SemaphoreType.DMA((2,)),
            pltpu.SemaphoreType.DMA((2,)),
        ],
    )
    return pl.pallas_call(
        _ffn_body,
        grid_spec=grid_spec,
        out_shape=jax.ShapeDtypeStruct((rows, D_MODEL), jnp.float32),
    )(item_expert, run_id, run_expert, msc,
      xg, W1, b1.reshape(N_EXPERTS, 1, D_FF),
      W2, b2.reshape(N_EXPERTS, 1, D_MODEL), w_rows)


def _sum3_body(yk_ref, o_ref):
    o_ref[...] = yk_ref[0] + yk_ref[1] + yk_ref[2]


def _sum3(yk3, n_tokens):
    blk = 256
    return pl.pallas_call(
        _sum3_body,
        grid=(n_tokens // blk,),
        in_specs=[pl.BlockSpec((3, blk, D_MODEL), lambda i: (0, i, 0))],
        out_specs=pl.BlockSpec((blk, D_MODEL), lambda i: (i, 0)),
        out_shape=jax.ShapeDtypeStruct((n_tokens, D_MODEL), jnp.float32),
    )(yk3)


def kernel(x, gate_w, W1, b1, W2, b2):
    B, L, D = x.shape
    N = B * L
    A = N * TOP_K
    NI = A // _T + N_EXPERTS
    P = NI * _T

    x2 = x.reshape(N, D)
    logits = x2 @ gate_w
    rn = jnp.arange(N)
    i1 = jnp.argmax(logits, axis=1).astype(jnp.int32)
    m1 = logits.at[rn, i1].set(-jnp.inf)
    i2 = jnp.argmax(m1, axis=1).astype(jnp.int32)
    m2 = m1.at[rn, i2].set(-jnp.inf)
    i3 = jnp.argmax(m2, axis=1).astype(jnp.int32)
    top_idx = jnp.stack([i1, i2, i3], axis=1)

    e_a = top_idx.reshape(A)
    oh = (e_a[:, None] == jnp.arange(N_EXPERTS, dtype=e_a.dtype)[None, :]
          ).astype(jnp.float32)
    ohb = oh.reshape(A // _T, _T, N_EXPERTS)
    tri = jnp.tril(jnp.ones((_T, _T), jnp.float32))
    cin = jnp.einsum("rs,bse->bre", tri, ohb)
    bsums = cin[:, _T - 1, :]
    bpre = jnp.cumsum(bsums, axis=0) - bsums
    csum = (cin + bpre[:, None, :]).reshape(A, N_EXPERTS)
    counts = (bpre[-1] + bsums[-1]).astype(jnp.int32)
    rank = (jnp.sum(csum * oh, axis=1)).astype(jnp.int32) - 1

    pcounts = ((counts + _T - 1) // _T) * _T
    pstarts = jnp.concatenate(
        [jnp.zeros(1, counts.dtype), jnp.cumsum(pcounts)[:-1]])
    ppos = (pstarts[e_a] + rank).astype(jnp.int32)
    tok = (jnp.arange(A, dtype=jnp.int32) // TOP_K)

    ts_p = (jnp.arange(P, dtype=jnp.int32) % N).at[ppos].set(tok)
    eg = jnp.asarray(EGYPTIAN, dtype=x.dtype)
    w_p = jnp.zeros(P, x.dtype).at[ppos].set(jnp.tile(eg, N))
    item_expert = (
        jnp.searchsorted(pstarts, jnp.arange(NI) * _T, side="right") - 1
    ).astype(jnp.int32)
    n_used = (jnp.sum(pcounts) // _T).astype(jnp.int32)
    last_e = item_expert[n_used - 1]
    iexp2 = jnp.where(jnp.arange(NI) < n_used, item_expert, last_e)
    change = jnp.concatenate([
        jnp.zeros(1, jnp.int32), (iexp2[1:] != iexp2[:-1]).astype(jnp.int32)])
    run_id = jnp.cumsum(change).astype(jnp.int32)
    run_expert = jnp.zeros(NI, jnp.int32).at[run_id].set(iexp2)
    n_runs = run_id[NI - 1] + 1
    msc = jnp.stack([n_used, n_runs]).astype(jnp.int32)
    cidx = ppos.reshape(N, TOP_K).T.reshape(A)

    xg = _gather_rows(x2, ts_p)
    y = _ffn(xg, W1, b1, W2, b2, w_p.reshape(P, 1),
             iexp2, run_id, run_expert, msc)
    yk = _gather_rows(y, cidx)
    out = _sum3(yk.reshape(TOP_K, N, D_MODEL), N)
    return out.reshape(B, L, D)


# speedup vs baseline: 1.0465x; 1.0465x over previous
import functools

import jax
import jax.numpy as jnp
from jax import lax
from jax.experimental import pallas as pl
from jax.experimental.pallas import tpu as pltpu
from jax.experimental.pallas import tpu_sc as plsc

D_MODEL = 1024
D_FF = 2048
N_EXPERTS = 24
TOP_K = 3
EGYPTIAN = (1.0 / 2.0, 1.0 / 3.0, 1.0 / 6.0)

_T = 128
_NW = 32
_CH = 96


def _gather_rows(table, idx):
    B = idx.shape[0]
    Dm = table.shape[1]
    bpw = B // _NW
    assert bpw % _CH == 0
    mesh = plsc.VectorSubcoreMesh(core_axis_name="c", subcore_axis_name="s")

    @functools.partial(
        pl.kernel,
        out_type=jax.ShapeDtypeStruct((B, Dm), jnp.float32),
        mesh=mesh,
        scratch_types=[
            pltpu.VMEM((_CH,), jnp.int32),
            pltpu.VMEM((_CH, Dm), jnp.float32),
            pltpu.SemaphoreType.DMA,
        ],
    )
    def gather_kernel(table_hbm, idx_hbm, out_hbm, idx_v, rows_v, sem):
        wid = lax.axis_index("s") * 2 + lax.axis_index("c")
        base = wid * bpw
        for c in range(bpw // _CH):
            off = base + c * _CH
            pltpu.sync_copy(idx_hbm.at[pl.ds(off, _CH)], idx_v)
            pltpu.async_copy(table_hbm.at[idx_v], rows_v, sem).wait()
            pltpu.sync_copy(rows_v, out_hbm.at[pl.ds(off, _CH)])

    return gather_kernel(table, idx)


def _ffn_body(iexp_ref, rid_ref, rexp_ref, msc_ref,
              xg_ref, w1_hbm, b1_ref, w2_hbm, b2_ref, wrow_ref, y_ref,
              w1buf, w2buf, w1b, w2b, sem1, sem2):
    i = pl.program_id(0)
    n_used = msc_ref[0]
    n_runs = msc_ref[1]

    @pl.when(i < n_used)
    def _():
        r = rid_ref[i]
        slot = lax.rem(r, 2)
        is_first = (i == 0) | (rid_ref[jnp.maximum(i - 1, 0)] != r)

        @pl.when(is_first)
        def _():
            @pl.when(i == 0)
            def _():
                e0 = rexp_ref[0]
                pltpu.make_async_copy(
                    w1_hbm.at[e0], w1buf.at[0], sem1.at[0]).start()
                pltpu.make_async_copy(
                    w2_hbm.at[e0], w2buf.at[0], sem2.at[0]).start()

            e = rexp_ref[r]
            pltpu.make_async_copy(
                w1_hbm.at[e], w1buf.at[slot], sem1.at[slot]).wait()
            pltpu.make_async_copy(
                w2_hbm.at[e], w2buf.at[slot], sem2.at[slot]).wait()

            @pl.when(r + 1 < n_runs)
            def _():
                en = rexp_ref[r + 1]
                ns = 1 - slot
                pltpu.make_async_copy(
                    w1_hbm.at[en], w1buf.at[ns], sem1.at[ns]).start()
                pltpu.make_async_copy(
                    w2_hbm.at[en], w2buf.at[ns], sem2.at[ns]).start()

            w1b[...] = w1buf[slot].astype(jnp.bfloat16)
            w2b[...] = w2buf[slot].astype(jnp.bfloat16)

        h = jnp.dot(xg_ref[...].astype(jnp.bfloat16), w1b[...],
                    preferred_element_type=jnp.float32)
        h = jnp.maximum(h + b1_ref[0], 0.0)
        y = jnp.dot(h.astype(jnp.bfloat16), w2b[...],
                    preferred_element_type=jnp.float32)
        y_ref[...] = (y + b2_ref[0]) * wrow_ref[...]


def _ffn(xg, W1, b1, W2, b2, w_rows, item_expert, run_id, run_expert, msc):
    rows = xg.shape[0]
    grid_spec = pltpu.PrefetchScalarGridSpec(
        num_scalar_prefetch=4,
        grid=(rows // _T,),
        in_specs=[
            pl.BlockSpec((_T, D_MODEL), lambda i, ie, ri, re, ms: (i, 0)),
            pl.BlockSpec(memory_space=pltpu.MemorySpace.HBM),
            pl.BlockSpec((1, 1, D_FF), lambda i, ie, ri, re, ms: (ie[i], 0, 0)),
            pl.BlockSpec(memory_space=pltpu.MemorySpace.HBM),
            pl.BlockSpec((1, 1, D_MODEL),
                         lambda i, ie, ri, re, ms: (ie[i], 0, 0)),
            pl.BlockSpec((_T, 1), lambda i, ie, ri, re, ms: (i, 0)),
        ],
        out_specs=pl.BlockSpec((_T, D_MODEL), lambda i, ie, ri, re, ms: (i, 0)),
        scratch_shapes=[
            pltpu.VMEM((2, D_MODEL, D_FF), jnp.float32),
            pltpu.VMEM((2, D_FF, D_MODEL), jnp.float32),
            pltpu.VMEM((D_MODEL, D_FF), jnp.bfloat16),
            pltpu.VMEM((D_FF, D_MODEL), jnp.bfloat16),
            pltpu.SemaphoreType.DMA((2,)),
            pltpu.SemaphoreType.DMA((2,)),
        ],
    )
    return pl.pallas_call(
        _ffn_body,
        grid_spec=grid_spec,
        out_shape=jax.ShapeDtypeStruct((rows, D_MODEL), jnp.float32),
    )(item_expert, run_id, run_expert, msc,
      xg, W1, b1.reshape(N_EXPERTS, 1, D_FF),
      W2, b2.reshape(N_EXPERTS, 1, D_MODEL), w_rows)


def _sum3_body(yk_ref, o_ref):
    o_ref[...] = yk_ref[0] + yk_ref[1] + yk_ref[2]


def _sum3(yk3, n_tokens):
    blk = 256
    return pl.pallas_call(
        _sum3_body,
        grid=(n_tokens // blk,),
        in_specs=[pl.BlockSpec((3, blk, D_MODEL), lambda i: (0, i, 0))],
        out_specs=pl.BlockSpec((blk, D_MODEL), lambda i: (i, 0)),
        out_shape=jax.ShapeDtypeStruct((n_tokens, D_MODEL), jnp.float32),
    )(yk3)


def kernel(x, gate_w, W1, b1, W2, b2):
    B, L, D = x.shape
    N = B * L
    A = N * TOP_K
    NI = A // _T + N_EXPERTS
    P = NI * _T

    x2 = x.reshape(N, D)
    logits = x2 @ gate_w
    top_idx = lax.top_k(logits, TOP_K)[1]

    e_a = top_idx.reshape(A)
    oh = (e_a[:, None] == jnp.arange(N_EXPERTS, dtype=e_a.dtype)[None, :]
          ).astype(jnp.float32)
    ohb = oh.reshape(A // _T, _T, N_EXPERTS)
    tri = jnp.tril(jnp.ones((_T, _T), jnp.float32))
    cin = jnp.einsum("rs,bse->bre", tri, ohb)
    bsums = cin[:, _T - 1, :]
    bpre = jnp.cumsum(bsums, axis=0) - bsums
    csum = (cin + bpre[:, None, :]).reshape(A, N_EXPERTS)
    counts = (bpre[-1] + bsums[-1]).astype(jnp.int32)
    rank = (jnp.sum(csum * oh, axis=1)).astype(jnp.int32) - 1

    pcounts = ((counts + _T - 1) // _T) * _T
    pstarts = jnp.concatenate(
        [jnp.zeros(1, counts.dtype), jnp.cumsum(pcounts)[:-1]])
    ppos = (pstarts[e_a] + rank).astype(jnp.int32)
    tok = (jnp.arange(A, dtype=jnp.int32) // TOP_K)

    ts_p = (jnp.arange(P, dtype=jnp.int32) % N).at[ppos].set(tok)
    eg = jnp.asarray(EGYPTIAN, dtype=x.dtype)
    w_p = jnp.zeros(P, x.dtype).at[ppos].set(jnp.tile(eg, N))
    item_expert = (
        jnp.searchsorted(pstarts, jnp.arange(NI) * _T, side="right") - 1
    ).astype(jnp.int32)
    n_used = (jnp.sum(pcounts) // _T).astype(jnp.int32)
    last_e = item_expert[n_used - 1]
    iexp2 = jnp.where(jnp.arange(NI) < n_used, item_expert, last_e)
    change = jnp.concatenate([
        jnp.zeros(1, jnp.int32), (iexp2[1:] != iexp2[:-1]).astype(jnp.int32)])
    run_id = jnp.cumsum(change).astype(jnp.int32)
    run_expert = jnp.zeros(NI, jnp.int32).at[run_id].set(iexp2)
    n_runs = run_id[NI - 1] + 1
    msc = jnp.stack([n_used, n_runs]).astype(jnp.int32)
    cidx = ppos.reshape(N, TOP_K).T.reshape(A)

    xg = _gather_rows(x2, ts_p)
    y = _ffn(xg, W1, b1, W2, b2, w_p.reshape(P, 1),
             iexp2, run_id, run_expert, msc)
    yk = _gather_rows(y, cidx)
    out = _sum3(yk.reshape(TOP_K, N, D_MODEL), N)
    return out.reshape(B, L, D)
